# baseline (speedup 1.0000x reference)
import functools

import jax
import jax.numpy as jnp
from jax import lax
from jax.experimental import pallas as pl
from jax.experimental.pallas import tpu as pltpu
from jax.experimental.pallas import tpu_sc as plsc

_EPS = 1e-08
_NC, _NS, _L = 2, 16, 16
_NW = _NC * _NS


def _sc_householder_body(x_hbm, i0_hbm, i1_hbm, pn_hbm, out_hbm,
                         x_v, pn_v, i0_v, i1_v):
    s, r = x_hbm.shape
    t_per_w = s // _NW
    n_groups = t_per_w // _L
    wid = lax.axis_index("c") * _NS + lax.axis_index("s")
    base = wid * t_per_w

    pltpu.sync_copy(x_hbm.at[pl.ds(base, t_per_w), :], x_v)
    pltpu.sync_copy(pn_hbm, pn_v)
    pltpu.sync_copy(i0_hbm.at[pl.ds(base, t_per_w)], i0_v)
    pltpu.sync_copy(i1_hbm.at[pl.ds(base, t_per_w)], i1_v)

    lanes = lax.iota(jnp.int32, _L)
    zero = jnp.zeros((_L,), jnp.float32)
    for g in range(n_groups):
        tok = lanes + g * _L
        i0 = i0_v[pl.ds(g * _L, _L)]
        i1 = i1_v[pl.ds(g * _L, _L)]

        def dot_body(k, carry):
            a, b, d, s0, s1 = carry
            kk = jnp.broadcast_to(k, (_L,))
            v0 = plsc.load_gather(pn_v, [i0, kk])
            v1 = plsc.load_gather(pn_v, [i1, kk])
            xr = plsc.load_gather(x_v, [tok, kk])
            return (a + v0 * xr, b + v1 * xr, d + v0 * v1,
                    s0 + v0 * v0, s1 + v1 * v1)

        a, b, d, s0, s1 = lax.fori_loop(
            0, r, dot_body, (zero, zero, zero, zero, zero))
        c0 = (2.0 * a) / (s0 + _EPS)
        c1 = (2.0 * (b - c0 * d)) / (s1 + _EPS)

        def upd_body(k, carry):
            kk = jnp.broadcast_to(k, (_L,))
            v0 = plsc.load_gather(pn_v, [i0, kk])
            v1 = plsc.load_gather(pn_v, [i1, kk])
            xr = plsc.load_gather(x_v, [tok, kk])
            plsc.store_scatter(x_v, [tok, kk], xr - c0 * v0 - c1 * v1)
            return carry

        lax.fori_loop(0, r, upd_body, 0)

    pltpu.sync_copy(x_v, out_hbm.at[pl.ds(base, t_per_w), :])


def _tc_proj_body(x2_ref, oi_ref, w_ref, out_ref):
    x2 = x2_ref[...]
    oi = oi_ref[...]
    s, r = x2.shape
    n_out = w_ref.shape[0] // r
    iota_e = lax.broadcasted_iota(jnp.int32, (s, n_out), 1)
    ohe = (oi == iota_e).astype(jnp.float32)
    xb = jnp.concatenate(
        [x2 * ohe[:, e:e + 1] for e in range(n_out)], axis=1)
    out_ref[...] = jnp.dot(xb, w_ref[...], preferred_element_type=jnp.float32)


def kernel(x, output_idx, process_indices, process_neurons, output_neurons):
    b, s, r = x.shape
    n_proc = process_neurons.shape[0]
    n_out, _, d_model = output_neurons.shape
    n_tok = b * s
    t_per_w = n_tok // _NW

    xs = x.reshape(n_tok, r)
    oi = output_idx.reshape(n_tok, 1).astype(jnp.int32)
    pi0 = process_indices[..., 0].reshape(n_tok).astype(jnp.int32)
    pi1 = process_indices[..., 1].reshape(n_tok).astype(jnp.int32)
    w = output_neurons.reshape(n_out * r, d_model)

    sc_house = functools.partial(
        pl.kernel,
        out_type=jax.ShapeDtypeStruct((n_tok, r), jnp.float32),
        mesh=plsc.VectorSubcoreMesh(core_axis_name="c", subcore_axis_name="s"),
        compiler_params=pltpu.CompilerParams(needs_layout_passes=False),
        scratch_types=[
            pltpu.VMEM((t_per_w, r), jnp.float32),
            pltpu.VMEM((n_proc, r), jnp.float32),
            pltpu.VMEM((t_per_w,), jnp.int32),
            pltpu.VMEM((t_per_w,), jnp.int32),
        ],
    )(_sc_householder_body)
    x2 = sc_house(xs, pi0, pi1, process_neurons)

    out = pl.pallas_call(
        _tc_proj_body,
        out_shape=jax.ShapeDtypeStruct((n_tok, d_model), jnp.float32),
    )(x2, oi, w)
    return out.reshape(b, s, d_model)
